# baseline (speedup 1.0000x reference)
import functools

import jax
import jax.numpy as jnp
from jax.experimental import pallas as pl
from jax.experimental.pallas import tpu as pltpu

_NBUF = 5
_BLK = 200


def _gcn_ring_kernel(scal_ref, adj_ref, x_ref, h0_ref, w_ref, o_ref,
                     abuf, sems, *, blk, nbuf, nsteps):
    theta = scal_ref[0]
    alpha = scal_ref[1]
    j = pl.program_id(0)
    nouter = nsteps // nbuf

    @pl.when(j == 0)
    def _prefetch():
        for k in range(nbuf):
            pltpu.make_async_copy(
                adj_ref.at[pl.ds(k * blk, blk), :], abuf.at[k], sems.at[k]
            ).start()

    for k in range(nbuf):
        b = j * nbuf + k
        pltpu.make_async_copy(
            adj_ref.at[pl.ds(b * blk, blk), :], abuf.at[k], sems.at[k]
        ).wait()
        hi = jnp.dot(abuf[k], x_ref[...], preferred_element_type=jnp.float32)
        row = b * blk
        support = (1.0 - alpha) * hi + alpha * h0_ref[pl.ds(row, blk), :]
        o_ref[pl.ds(k * blk, blk), :] = (
            theta * jnp.dot(support, w_ref[...], preferred_element_type=jnp.float32)
            + (1.0 - theta) * support
            + x_ref[pl.ds(row, blk), :]
        )

        @pl.when(j < nouter - 1)
        def _next():
            nb = (j + 1) * nbuf + k
            pltpu.make_async_copy(
                adj_ref.at[pl.ds(nb * blk, blk), :], abuf.at[k], sems.at[k]
            ).start()


@jax.jit
def kernel(x, adj, h0, W, lamda, alpha, layer_idx):
    n, d = x.shape
    blk, nbuf = _BLK, _NBUF
    nsteps = n // blk
    theta = jnp.log(
        jnp.asarray(lamda, jnp.float32) / jnp.asarray(layer_idx, jnp.float32) + 1.0
    )
    scal = jnp.stack([theta, jnp.asarray(alpha, jnp.float32)])

    grid = (nsteps // nbuf,)
    out = pl.pallas_call(
        functools.partial(_gcn_ring_kernel, blk=blk, nbuf=nbuf, nsteps=nsteps),
        grid=grid,
        in_specs=[
            pl.BlockSpec(memory_space=pltpu.SMEM),
            pl.BlockSpec(memory_space=pltpu.MemorySpace.HBM),
            pl.BlockSpec((n, d), lambda j: (0, 0)),
            pl.BlockSpec((n, d), lambda j: (0, 0)),
            pl.BlockSpec((d, d), lambda j: (0, 0)),
        ],
        out_specs=pl.BlockSpec((blk * nbuf, d), lambda j: (j, 0)),
        out_shape=jax.ShapeDtypeStruct((n, d), jnp.float32),
        scratch_shapes=[
            pltpu.VMEM((nbuf, blk, n), jnp.float32),
            pltpu.SemaphoreType.DMA((nbuf,)),
        ],
        compiler_params=pltpu.CompilerParams(
            dimension_semantics=("arbitrary",),
        ),
    )(scal, adj, x, h0, W)
    return out
